# baseline (speedup 1.0000x reference)
import functools

import jax
import jax.numpy as jnp
from jax import lax
from jax.experimental import pallas as pl
from jax.experimental.pallas import tpu as pltpu
from jax.experimental.pallas import tpu_sc as plsc

B, L, H, SPAN = 32, 256, 256, 4
NC, NS = 2, 16

_CONTRACT_00 = (((0,), (0,)), ((), ()))
_CONTRACT_01 = (((0,), (1,)), ((), ()))


def _dg(a, b, dims):
    return lax.dot_general(a, b, dims, preferred_element_type=jnp.float32)


def _scatter_chunk(rows_buf, hl_v, acc_v, base):
    lane = lax.iota(jnp.int32, 16)
    for g in range(8):
        hvec = hl_v[pl.ds(base + g * 16, 16)]
        jvec = lane + (g * 16)

        @plsc.parallel_loop(0, H, step=4, unroll=4)
        def _(ci, hvec=hvec, jvec=jvec):
            for u in range(4):
                cvec = jnp.full((16,), ci + u, dtype=jnp.int32)
                vals = plsc.load_gather(rows_buf, [cvec, jvec])
                plsc.addupdate_scatter(acc_v, [cvec, hvec], vals)


def _sc1_body(yfT_hbm, ybflat_hbm, hl_hbm, zfT_hbm, gb_hbm,
              rows_v, gdst_v, hl_v, hg_v, acc_v, sem_i, sem_r, sem_h, sem_g):
    c = lax.axis_index("c")
    s = lax.axis_index("s")
    b = s * NC + c
    cp_i = pltpu.async_copy(yfT_hbm.at[pl.ds(b * H, H)], acc_v, sem_i)
    cp_r = pltpu.async_copy(
        yfT_hbm.at[pl.ds(b * H, H), pl.ds(0, 128)], rows_v, sem_r)
    cp_h = pltpu.async_copy(hl_hbm.at[b], hl_v, sem_h)
    cp_h.wait()
    boff = b * L
    for t in range(L // 16):
        hg_v[pl.ds(t * 16, 16)] = hl_v[pl.ds(t * 16, 16)] + boff
    g_cp = pltpu.async_copy(ybflat_hbm.at[hg_v.at[pl.ds(0, 64)]], gdst_v, sem_g)
    cp_i.wait()
    cp_r.wait()
    _scatter_chunk(rows_v, hl_v, acc_v, 0)
    cp_r = pltpu.async_copy(
        yfT_hbm.at[pl.ds(b * H, H), pl.ds(128, 128)], rows_v, sem_r)
    g_cp.wait()
    pltpu.sync_copy(gdst_v, gb_hbm.at[b].at[pl.ds(0, 64)])
    g_cp = pltpu.async_copy(
        ybflat_hbm.at[hg_v.at[pl.ds(64, 64)]], gdst_v, sem_g)
    cp_r.wait()
    _scatter_chunk(rows_v, hl_v, acc_v, 128)
    for k in (1, 2, 3):
        g_cp.wait()
        pltpu.sync_copy(gdst_v, gb_hbm.at[b].at[pl.ds(k * 64, 64)])
        if k < 3:
            g_cp = pltpu.async_copy(
                ybflat_hbm.at[hg_v.at[pl.ds((k + 1) * 64, 64)]], gdst_v, sem_g)
    pltpu.sync_copy(acc_v, zfT_hbm.at[pl.ds(b * H, H)])


_sc1 = functools.partial(
    pl.kernel,
    out_type=[jax.ShapeDtypeStruct((B * H, L), jnp.float32),
              jax.ShapeDtypeStruct((B, L, H), jnp.float32)],
    mesh=plsc.VectorSubcoreMesh(core_axis_name="c", subcore_axis_name="s",
                                num_cores=NC, num_subcores=NS),
    compiler_params=pltpu.CompilerParams(
        use_tc_tiling_on_sc=True, needs_layout_passes=False),
    scratch_types=[
        pltpu.VMEM((H, 128), jnp.float32),
        pltpu.VMEM((64, H), jnp.float32),
        pltpu.VMEM((L,), jnp.int32),
        pltpu.VMEM((L,), jnp.int32),
        pltpu.VMEM((H, L), jnp.float32),
        pltpu.SemaphoreType.DMA,
        pltpu.SemaphoreType.DMA,
        pltpu.SemaphoreType.DMA,
        pltpu.SemaphoreType.DMA,
    ],
)(_sc1_body)


def _sc2_body(yfT_hbm, ybflat_hbm, hl_hbm, spanrep_hbm, ybspan_hbm, ybhead_hbm,
              out_hbm, rows_v, hl_v, spanrep_v, ybspan_v, ybhead_v,
              yba_v, ybb_v, out_v, acc_v,
              sem_i, sem_r, sem_h, sem_s, sem_ga, sem_gb):
    c = lax.axis_index("c")
    s = lax.axis_index("s")
    b = s * NC + c
    cp_i = pltpu.async_copy(yfT_hbm.at[pl.ds(b * H, H)], acc_v, sem_i)
    cp_r = pltpu.async_copy(
        yfT_hbm.at[pl.ds(b * H, H), pl.ds(0, 128)], rows_v, sem_r)
    cp_h = pltpu.async_copy(hl_hbm.at[b], hl_v, sem_h)
    cp_s1 = pltpu.async_copy(ybspan_hbm.at[b], ybspan_v, sem_s)
    cp_s2 = pltpu.async_copy(ybhead_hbm.at[b], ybhead_v, sem_s)
    cp_sp = pltpu.async_copy(spanrep_hbm.at[b], spanrep_v, sem_s)
    cp_s1.wait()
    cp_s2.wait()
    g_a = pltpu.async_copy(ybflat_hbm.at[ybspan_v], yba_v, sem_ga)
    g_b = pltpu.async_copy(ybflat_hbm.at[ybhead_v], ybb_v, sem_gb)
    cp_h.wait()
    cp_i.wait()
    cp_r.wait()
    _scatter_chunk(rows_v, hl_v, acc_v, 0)
    cp_r = pltpu.async_copy(
        yfT_hbm.at[pl.ds(b * H, H), pl.ds(128, 128)], rows_v, sem_r)
    cp_r.wait()
    _scatter_chunk(rows_v, hl_v, acc_v, 128)
    lane = lax.iota(jnp.int32, 16)
    cp_sp.wait()
    for g in range(16):
        hvec = lane + (g * 16)
        for half in range(2):
            tot = jnp.zeros((16,), jnp.float32)
            for r in range(SPAN):
                cvec = spanrep_v[half * SPAN + r]
                vals = plsc.load_gather(acc_v, [hvec, cvec])
                tot = tot + jnp.maximum(vals, 0.0)
            out_v[pl.ds(half * 512 + g * 16, 16)] = tot * (1.0 / SPAN)
    g_a.wait()
    g_b.wait()
    for cg in range(16):
        sl = pl.ds(cg * 16, 16)
        for half in range(2):
            tot = jnp.zeros((16,), jnp.float32)
            for r in range(SPAN):
                row = half * SPAN + r
                tot = tot + jnp.maximum(yba_v[row, sl] + ybb_v[row, sl], 0.0)
            out_v[pl.ds(half * 512 + 256 + cg * 16, 16)] = tot * (1.0 / SPAN)
    pltpu.sync_copy(out_v, out_hbm.at[b])


_sc2 = functools.partial(
    pl.kernel,
    out_type=jax.ShapeDtypeStruct((B, 4 * H), jnp.float32),
    mesh=plsc.VectorSubcoreMesh(core_axis_name="c", subcore_axis_name="s",
                                num_cores=NC, num_subcores=NS),
    compiler_params=pltpu.CompilerParams(
        use_tc_tiling_on_sc=True, needs_layout_passes=False),
    scratch_types=[
        pltpu.VMEM((H, 128), jnp.float32),
        pltpu.VMEM((L,), jnp.int32),
        pltpu.VMEM((2 * SPAN, 16), jnp.int32),
        pltpu.VMEM((2 * SPAN,), jnp.int32),
        pltpu.VMEM((2 * SPAN,), jnp.int32),
        pltpu.VMEM((2 * SPAN, H), jnp.float32),
        pltpu.VMEM((2 * SPAN, H), jnp.float32),
        pltpu.VMEM((4 * H,), jnp.float32),
        pltpu.VMEM((H, L), jnp.float32),
        pltpu.SemaphoreType.DMA,
        pltpu.SemaphoreType.DMA,
        pltpu.SemaphoreType.DMA,
        pltpu.SemaphoreType.DMA,
        pltpu.SemaphoreType.DMA,
        pltpu.SemaphoreType.DMA,
    ],
)(_sc2_body)


def _tc1_body(x_ref, wf_ref, wb_ref, bfc_ref, bb_ref, yfT_ref, yb_ref):
    x = x_ref[0]
    yfT_ref[...] = _dg(wf_ref[...], x, _CONTRACT_01) + bfc_ref[...]
    yb_ref[...] = jnp.dot(x, wb_ref[...],
                          preferred_element_type=jnp.float32) + bb_ref[...]


def _tc2_body(zfT_ref, yb_ref, gb_ref, wft_ref, wfb_ref, wbt_ref, wbb_ref,
              bfc_ref, bb_ref, yfT_ref, yb_ref_out):
    xfT = jnp.maximum(zfT_ref[...], 0.0)
    xb = jnp.maximum(yb_ref[...] + gb_ref[0], 0.0)
    yfT_ref[...] = (_dg(wft_ref[...], xfT, _CONTRACT_00)
                    + _dg(wfb_ref[...], xb, _CONTRACT_01) + bfc_ref[...])
    yb_ref_out[...] = (_dg(xfT, wbt_ref[...], _CONTRACT_00)
                       + jnp.dot(xb, wbb_ref[...],
                                 preferred_element_type=jnp.float32)
                       + bb_ref[...])


def _batch_spec(shape_tail):
    return pl.BlockSpec((1,) + shape_tail, lambda b: (b,) + (0,) * len(shape_tail))


def _full_spec(shape):
    return pl.BlockSpec(shape, lambda b: (0,) * len(shape))


def _row_spec(n_cols):
    return pl.BlockSpec((L, n_cols), lambda b: (b, 0))


def _tc_call(body, ins, weights, biases):
    act_specs = []
    for s in ins:
        if s.ndim == 3:
            act_specs.append(_batch_spec(s.shape[1:]))
        elif s.shape[0] == B * H:
            act_specs.append(pl.BlockSpec((H, L), lambda b: (b, 0)))
        else:
            act_specs.append(_row_spec(s.shape[-1]))
    w_specs = [_full_spec(w.shape) for w in weights] + \
              [_full_spec(bi.shape) for bi in biases]
    return pl.pallas_call(
        body,
        grid=(B,),
        in_specs=act_specs + w_specs,
        out_specs=[pl.BlockSpec((H, L), lambda b: (b, 0)), _row_spec(H)],
        out_shape=[jax.ShapeDtypeStruct((B * H, L), jnp.float32),
                   jax.ShapeDtypeStruct((B * L, H), jnp.float32)],
        compiler_params=pltpu.CompilerParams(dimension_semantics=("arbitrary",)),
    )(*ins, *weights, *biases)


@jax.jit
def kernel(word_h, heads, types, instances,
           W_fw0, W_bw0, W_fw1, W_bw1, b_fw0, b_bw0, b_fw1, b_bw1):
    del types
    heads_i = heads.astype(jnp.int32)
    b_off = jnp.arange(B, dtype=jnp.int32)[:, None] * L
    inst = instances.astype(jnp.int32)
    offs = 1 + jnp.arange(SPAN, dtype=jnp.int32)
    span_local = jnp.concatenate(
        [inst[:, 0:1] + offs[None, :], inst[:, 2:3] + offs[None, :]], axis=1)
    spanrep = jnp.broadcast_to(span_local[:, :, None], (B, 2 * SPAN, 16))
    ybspan_idx = b_off + span_local
    ybhead_idx = b_off + jnp.take_along_axis(heads_i, span_local, axis=1)

    yfT0, yb0 = _tc_call(_tc1_body, [word_h], [W_fw0, W_bw0],
                         [b_fw0.reshape(H, 1), b_bw0.reshape(1, H)])
    zfT0, gb0 = _sc1(yfT0, yb0, heads_i)
    yfT1, yb1 = _tc_call(
        _tc2_body, [zfT0, yb0, gb0],
        [W_fw1[:H], W_fw1[H:], W_bw1[:H], W_bw1[H:]],
        [b_fw1.reshape(H, 1), b_bw1.reshape(1, H)])
    return _sc2(yfT1, yb1, heads_i, spanrep, ybspan_idx, ybhead_idx)


# speedup vs baseline: 1.0095x; 1.0095x over previous
import functools

import jax
import jax.numpy as jnp
from jax import lax
from jax.experimental import pallas as pl
from jax.experimental.pallas import tpu as pltpu
from jax.experimental.pallas import tpu_sc as plsc

B, L, H, SPAN = 32, 256, 256, 4
NC, NS = 2, 16

_CONTRACT_00 = (((0,), (0,)), ((), ()))
_CONTRACT_01 = (((0,), (1,)), ((), ()))


def _dg(a, b, dims):
    return lax.dot_general(a, b, dims, preferred_element_type=jnp.float32)


def _scatter_chunk(rows_buf, hl_v, acc_v, base):
    lane = lax.iota(jnp.int32, 16)
    for g in range(8):
        hvec = hl_v[pl.ds(base + g * 16, 16)]
        jvec = lane + (g * 16)

        @plsc.parallel_loop(0, H, step=4, unroll=4)
        def _(ci, hvec=hvec, jvec=jvec):
            for u in range(4):
                cvec = jnp.full((16,), ci + u, dtype=jnp.int32)
                vals = plsc.load_gather(rows_buf, [cvec, jvec])
                plsc.addupdate_scatter(acc_v, [cvec, hvec], vals)


def _sc1_body(yfT_hbm, ybflat_hbm, hl_hbm, zfT_hbm, gb_hbm,
              rows_v, gdst_v, hl_v, hg_v, acc_v, sem_i, sem_r, sem_h, sem_g):
    c = lax.axis_index("c")
    s = lax.axis_index("s")
    b = s * NC + c
    cp_i = pltpu.async_copy(yfT_hbm.at[pl.ds(b * H, H)], acc_v, sem_i)
    cp_r = pltpu.async_copy(
        yfT_hbm.at[pl.ds(b * H, H), pl.ds(0, 128)], rows_v, sem_r)
    cp_h = pltpu.async_copy(hl_hbm.at[b], hl_v, sem_h)
    cp_h.wait()
    boff = b * L
    for t in range(L // 16):
        hg_v[pl.ds(t * 16, 16)] = hl_v[pl.ds(t * 16, 16)] + boff
    g_cp = pltpu.async_copy(ybflat_hbm.at[hg_v.at[pl.ds(0, 64)]], gdst_v, sem_g)
    cp_i.wait()
    cp_r.wait()
    _scatter_chunk(rows_v, hl_v, acc_v, 0)
    cp_r = pltpu.async_copy(
        yfT_hbm.at[pl.ds(b * H, H), pl.ds(128, 128)], rows_v, sem_r)
    g_cp.wait()
    pltpu.sync_copy(gdst_v, gb_hbm.at[b].at[pl.ds(0, 64)])
    g_cp = pltpu.async_copy(
        ybflat_hbm.at[hg_v.at[pl.ds(64, 64)]], gdst_v, sem_g)
    cp_r.wait()
    _scatter_chunk(rows_v, hl_v, acc_v, 128)
    z_cp = pltpu.async_copy(acc_v, zfT_hbm.at[pl.ds(b * H, H)], sem_i)
    for k in (1, 2, 3):
        g_cp.wait()
        pltpu.sync_copy(gdst_v, gb_hbm.at[b].at[pl.ds(k * 64, 64)])
        if k < 3:
            g_cp = pltpu.async_copy(
                ybflat_hbm.at[hg_v.at[pl.ds((k + 1) * 64, 64)]], gdst_v, sem_g)
    z_cp.wait()


_sc1 = functools.partial(
    pl.kernel,
    out_type=[jax.ShapeDtypeStruct((B * H, L), jnp.float32),
              jax.ShapeDtypeStruct((B, L, H), jnp.float32)],
    mesh=plsc.VectorSubcoreMesh(core_axis_name="c", subcore_axis_name="s",
                                num_cores=NC, num_subcores=NS),
    compiler_params=pltpu.CompilerParams(
        use_tc_tiling_on_sc=True, needs_layout_passes=False),
    scratch_types=[
        pltpu.VMEM((H, 128), jnp.float32),
        pltpu.VMEM((64, H), jnp.float32),
        pltpu.VMEM((L,), jnp.int32),
        pltpu.VMEM((L,), jnp.int32),
        pltpu.VMEM((H, L), jnp.float32),
        pltpu.SemaphoreType.DMA,
        pltpu.SemaphoreType.DMA,
        pltpu.SemaphoreType.DMA,
        pltpu.SemaphoreType.DMA,
    ],
)(_sc1_body)


def _sc2_body(yfT_hbm, ybflat_hbm, hl_hbm, spanrep_hbm, ybspan_hbm, ybhead_hbm,
              out_hbm, rows_v, hl_v, spanrep_v, ybspan_v, ybhead_v,
              yba_v, ybb_v, out_v, acc_v,
              sem_i, sem_r, sem_h, sem_s, sem_ga, sem_gb):
    c = lax.axis_index("c")
    s = lax.axis_index("s")
    b = s * NC + c
    cp_i = pltpu.async_copy(yfT_hbm.at[pl.ds(b * H, H)], acc_v, sem_i)
    cp_r = pltpu.async_copy(
        yfT_hbm.at[pl.ds(b * H, H), pl.ds(0, 128)], rows_v, sem_r)
    cp_h = pltpu.async_copy(hl_hbm.at[b], hl_v, sem_h)
    cp_s1 = pltpu.async_copy(ybspan_hbm.at[b], ybspan_v, sem_s)
    cp_s2 = pltpu.async_copy(ybhead_hbm.at[b], ybhead_v, sem_s)
    cp_sp = pltpu.async_copy(spanrep_hbm.at[b], spanrep_v, sem_s)
    cp_s1.wait()
    cp_s2.wait()
    g_a = pltpu.async_copy(ybflat_hbm.at[ybspan_v], yba_v, sem_ga)
    g_b = pltpu.async_copy(ybflat_hbm.at[ybhead_v], ybb_v, sem_gb)
    cp_h.wait()
    cp_i.wait()
    cp_r.wait()
    _scatter_chunk(rows_v, hl_v, acc_v, 0)
    cp_r = pltpu.async_copy(
        yfT_hbm.at[pl.ds(b * H, H), pl.ds(128, 128)], rows_v, sem_r)
    cp_r.wait()
    _scatter_chunk(rows_v, hl_v, acc_v, 128)
    lane = lax.iota(jnp.int32, 16)
    cp_sp.wait()
    for g in range(16):
        hvec = lane + (g * 16)
        for half in range(2):
            tot = jnp.zeros((16,), jnp.float32)
            for r in range(SPAN):
                cvec = spanrep_v[half * SPAN + r]
                vals = plsc.load_gather(acc_v, [hvec, cvec])
                tot = tot + jnp.maximum(vals, 0.0)
            out_v[pl.ds(half * 512 + g * 16, 16)] = tot * (1.0 / SPAN)
    g_a.wait()
    g_b.wait()
    for cg in range(16):
        sl = pl.ds(cg * 16, 16)
        for half in range(2):
            tot = jnp.zeros((16,), jnp.float32)
            for r in range(SPAN):
                row = half * SPAN + r
                tot = tot + jnp.maximum(yba_v[row, sl] + ybb_v[row, sl], 0.0)
            out_v[pl.ds(half * 512 + 256 + cg * 16, 16)] = tot * (1.0 / SPAN)
    pltpu.sync_copy(out_v, out_hbm.at[b])


_sc2 = functools.partial(
    pl.kernel,
    out_type=jax.ShapeDtypeStruct((B, 4 * H), jnp.float32),
    mesh=plsc.VectorSubcoreMesh(core_axis_name="c", subcore_axis_name="s",
                                num_cores=NC, num_subcores=NS),
    compiler_params=pltpu.CompilerParams(
        use_tc_tiling_on_sc=True, needs_layout_passes=False),
    scratch_types=[
        pltpu.VMEM((H, 128), jnp.float32),
        pltpu.VMEM((L,), jnp.int32),
        pltpu.VMEM((2 * SPAN, 16), jnp.int32),
        pltpu.VMEM((2 * SPAN,), jnp.int32),
        pltpu.VMEM((2 * SPAN,), jnp.int32),
        pltpu.VMEM((2 * SPAN, H), jnp.float32),
        pltpu.VMEM((2 * SPAN, H), jnp.float32),
        pltpu.VMEM((4 * H,), jnp.float32),
        pltpu.VMEM((H, L), jnp.float32),
        pltpu.SemaphoreType.DMA,
        pltpu.SemaphoreType.DMA,
        pltpu.SemaphoreType.DMA,
        pltpu.SemaphoreType.DMA,
        pltpu.SemaphoreType.DMA,
        pltpu.SemaphoreType.DMA,
    ],
)(_sc2_body)


def _tc1_body(x_ref, wf_ref, wb_ref, bfc_ref, bb_ref, yfT_ref, yb_ref):
    x = x_ref[0]
    yfT_ref[...] = _dg(wf_ref[...], x, _CONTRACT_01) + bfc_ref[...]
    yb_ref[...] = jnp.dot(x, wb_ref[...],
                          preferred_element_type=jnp.float32) + bb_ref[...]


def _tc2_body(zfT_ref, yb_ref, gb_ref, wft_ref, wfb_ref, wbt_ref, wbb_ref,
              bfc_ref, bb_ref, yfT_ref, yb_ref_out):
    xfT = jnp.maximum(zfT_ref[...], 0.0)
    xb = jnp.maximum(yb_ref[...] + gb_ref[0], 0.0)
    yfT_ref[...] = (_dg(wft_ref[...], xfT, _CONTRACT_00)
                    + _dg(wfb_ref[...], xb, _CONTRACT_01) + bfc_ref[...])
    yb_ref_out[...] = (_dg(xfT, wbt_ref[...], _CONTRACT_00)
                       + jnp.dot(xb, wbb_ref[...],
                                 preferred_element_type=jnp.float32)
                       + bb_ref[...])


def _batch_spec(shape_tail):
    return pl.BlockSpec((1,) + shape_tail, lambda b: (b,) + (0,) * len(shape_tail))


def _full_spec(shape):
    return pl.BlockSpec(shape, lambda b: (0,) * len(shape))


def _row_spec(n_cols):
    return pl.BlockSpec((L, n_cols), lambda b: (b, 0))


def _tc_call(body, ins, weights, biases):
    act_specs = []
    for s in ins:
        if s.ndim == 3:
            act_specs.append(_batch_spec(s.shape[1:]))
        elif s.shape[0] == B * H:
            act_specs.append(pl.BlockSpec((H, L), lambda b: (b, 0)))
        else:
            act_specs.append(_row_spec(s.shape[-1]))
    w_specs = [_full_spec(w.shape) for w in weights] + \
              [_full_spec(bi.shape) for bi in biases]
    return pl.pallas_call(
        body,
        grid=(B,),
        in_specs=act_specs + w_specs,
        out_specs=[pl.BlockSpec((H, L), lambda b: (b, 0)), _row_spec(H)],
        out_shape=[jax.ShapeDtypeStruct((B * H, L), jnp.float32),
                   jax.ShapeDtypeStruct((B * L, H), jnp.float32)],
        compiler_params=pltpu.CompilerParams(dimension_semantics=("arbitrary",)),
    )(*ins, *weights, *biases)


@jax.jit
def kernel(word_h, heads, types, instances,
           W_fw0, W_bw0, W_fw1, W_bw1, b_fw0, b_bw0, b_fw1, b_bw1):
    del types
    heads_i = heads.astype(jnp.int32)
    b_off = jnp.arange(B, dtype=jnp.int32)[:, None] * L
    inst = instances.astype(jnp.int32)
    offs = 1 + jnp.arange(SPAN, dtype=jnp.int32)
    span_local = jnp.concatenate(
        [inst[:, 0:1] + offs[None, :], inst[:, 2:3] + offs[None, :]], axis=1)
    spanrep = jnp.broadcast_to(span_local[:, :, None], (B, 2 * SPAN, 16))
    ybspan_idx = b_off + span_local
    ybhead_idx = b_off + jnp.take_along_axis(heads_i, span_local, axis=1)

    yfT0, yb0 = _tc_call(_tc1_body, [word_h], [W_fw0, W_bw0],
                         [b_fw0.reshape(H, 1), b_bw0.reshape(1, H)])
    zfT0, gb0 = _sc1(yfT0, yb0, heads_i)
    yfT1, yb1 = _tc_call(
        _tc2_body, [zfT0, yb0, gb0],
        [W_fw1[:H], W_fw1[H:], W_bw1[:H], W_bw1[H:]],
        [b_fw1.reshape(H, 1), b_bw1.reshape(1, H)])
    return _sc2(yfT1, yb1, heads_i, spanrep, ybspan_idx, ybhead_idx)
